# Initial kernel scaffold; baseline (speedup 1.0000x reference)
import jax
import jax.numpy as jnp
from jax.experimental import pallas as pl


def kernel(word_ids, topic_ids, act_ids, word_table, topic_table, act_table):
    raise NotImplementedError("write your pallas kernel here")


# speedup vs baseline: 4.1427x; 4.1427x over previous
import functools

import jax
import jax.numpy as jnp
from jax import lax
from jax.experimental import pallas as pl
from jax.experimental.pallas import tpu as pltpu
from jax.experimental.pallas import tpu_sc as plsc

WORD_VOCAB = 100000
TOPIC_VOCAB = 1000
ACT_VOCAB = 1000
WORD_D = 64
TOPIC_D = 32
ACT_D = 32
B = 4096
L = 200

_NC = 2
_NS = 16
_NW = _NC * _NS

_TOTAL_W = B * L
_W_PER = _TOTAL_W // _NW
_GCHUNK = 128
_KINFLIGHT = 8
_SUPER = _GCHUNK * _KINFLIGHT
_NSUPER = _W_PER // _SUPER
_S_PER = B // _NW


def _body(word_ids, topic_ids, act_ids, word_tab, topic_tab, act_tab,
          word_out, topic_out, act_out,
          idx_v, rows_v, tidx_v, trows_v, aidx_v, arows_v, sem):
  c = lax.axis_index("c")
  s = lax.axis_index("s")
  wid = s * _NC + c

  row_base = wid * (_W_PER // _GCHUNK)
  out_base = wid * _W_PER

  @pl.loop(0, _NSUPER)
  def _w(ci):
    pltpu.sync_copy(word_ids.at[pl.ds(row_base + ci * _KINFLIGHT, _KINFLIGHT)],
                    idx_v)
    descs = []
    for j in range(_KINFLIGHT):
      descs.append(pltpu.async_copy(
          word_tab.at[idx_v.at[j]],
          rows_v.at[pl.ds(j * _GCHUNK, _GCHUNK)],
          sem))
    for d in descs:
      d.wait()
    pltpu.sync_copy(rows_v, word_out.at[pl.ds(out_base + ci * _SUPER, _SUPER)])

  sb = wid * _S_PER
  pltpu.sync_copy(topic_ids.at[pl.ds(sb, _S_PER)], tidx_v)
  pltpu.sync_copy(act_ids.at[pl.ds(sb, _S_PER)], aidx_v)
  dt = pltpu.async_copy(topic_tab.at[tidx_v], trows_v, sem)
  da = pltpu.async_copy(act_tab.at[aidx_v], arows_v, sem)
  dt.wait()
  da.wait()
  pltpu.sync_copy(trows_v, topic_out.at[pl.ds(sb, _S_PER)])
  pltpu.sync_copy(arows_v, act_out.at[pl.ds(sb, _S_PER)])


@jax.jit
def _run(word_ids2d, topic_ids, act_ids, word_table, topic_table, act_table):
  mesh = plsc.VectorSubcoreMesh(core_axis_name="c", subcore_axis_name="s")
  k = pl.kernel(
      _body,
      out_type=(
          jax.ShapeDtypeStruct((_TOTAL_W, WORD_D), jnp.float32),
          jax.ShapeDtypeStruct((B, TOPIC_D), jnp.float32),
          jax.ShapeDtypeStruct((B, ACT_D), jnp.float32),
      ),
      mesh=mesh,
      scratch_types=(
          pltpu.VMEM((_KINFLIGHT, _GCHUNK), jnp.int32),
          pltpu.VMEM((_SUPER, WORD_D), jnp.float32),
          pltpu.VMEM((_S_PER,), jnp.int32),
          pltpu.VMEM((_S_PER, TOPIC_D), jnp.float32),
          pltpu.VMEM((_S_PER,), jnp.int32),
          pltpu.VMEM((_S_PER, ACT_D), jnp.float32),
          pltpu.SemaphoreType.DMA,
      ),
      compiler_params=pltpu.CompilerParams(use_tc_tiling_on_sc=False),
  )
  return k(word_ids2d, topic_ids, act_ids, word_table, topic_table, act_table)


def kernel(word_ids, topic_ids, act_ids, word_table, topic_table, act_table):
  word_ids2d = word_ids.reshape(_TOTAL_W // _GCHUNK, _GCHUNK).astype(jnp.int32)
  wout, tout, aout = _run(word_ids2d,
                          topic_ids.astype(jnp.int32),
                          act_ids.astype(jnp.int32),
                          word_table, topic_table, act_table)
  return (wout.reshape(B, L, WORD_D), tout, aout)


# speedup vs baseline: 4.2434x; 1.0243x over previous
import functools

import jax
import jax.numpy as jnp
from jax import lax
from jax.experimental import pallas as pl
from jax.experimental.pallas import tpu as pltpu
from jax.experimental.pallas import tpu_sc as plsc

WORD_VOCAB = 100000
TOPIC_VOCAB = 1000
ACT_VOCAB = 1000
WORD_D = 64
TOPIC_D = 32
ACT_D = 32
B = 4096
L = 200

_NC = 2
_NS = 16
_NW = _NC * _NS

_TOTAL_W = B * L
_W_PER = _TOTAL_W // _NW
_GCHUNK = 128
_KINFLIGHT = 4
_SUPER = _GCHUNK * _KINFLIGHT
_NSUPER = _W_PER // _SUPER
_ROWS_PER_W = _W_PER // _GCHUNK
_S_PER = B // _NW


def _body(word_ids, topic_ids, act_ids, word_tab, topic_tab, act_tab,
          word_out, topic_out, act_out,
          idx_all, buf_a, buf_b, tidx_v, trows_v, aidx_v, arows_v,
          gsem, osem, tsem):
  c = lax.axis_index("c")
  s = lax.axis_index("s")
  wid = s * _NC + c

  sb = wid * _S_PER
  pltpu.sync_copy(topic_ids.at[pl.ds(sb, _S_PER)], tidx_v)
  pltpu.sync_copy(act_ids.at[pl.ds(sb, _S_PER)], aidx_v)
  dt = pltpu.async_copy(topic_tab.at[tidx_v], trows_v, tsem)
  da = pltpu.async_copy(act_tab.at[aidx_v], arows_v, tsem)

  row_base = wid * _ROWS_PER_W
  out_base = wid * _W_PER
  pltpu.sync_copy(word_ids.at[pl.ds(row_base, _ROWS_PER_W)], idx_all)

  def fire(i, buf):
    return [pltpu.async_copy(
        word_tab.at[idx_all.at[i * _KINFLIGHT + j]],
        buf.at[pl.ds(j * _GCHUNK, _GCHUNK)],
        gsem) for j in range(_KINFLIGHT)]

  def out_start(i, buf):
    pltpu.async_copy(buf, word_out.at[pl.ds(out_base + i * _SUPER, _SUPER)],
                     osem)

  def out_wait(buf):
    pltpu.make_async_copy(
        buf, word_out.at[pl.ds(out_base, _SUPER)], osem).wait()

  for i, buf in ((0, buf_a), (1, buf_b)):
    for d in fire(i, buf):
      d.wait()
    out_start(i, buf)

  @pl.loop(1, _NSUPER // 2)
  def _w(ci):
    for b, buf in ((0, buf_a), (1, buf_b)):
      i = ci * 2 + b
      out_wait(buf)
      for d in fire(i, buf):
        d.wait()
      out_start(i, buf)

  out_wait(buf_a)
  out_wait(buf_b)

  dt.wait()
  da.wait()
  pltpu.sync_copy(trows_v, topic_out.at[pl.ds(sb, _S_PER)])
  pltpu.sync_copy(arows_v, act_out.at[pl.ds(sb, _S_PER)])


@jax.jit
def _run(word_ids2d, topic_ids, act_ids, word_table, topic_table, act_table):
  mesh = plsc.VectorSubcoreMesh(core_axis_name="c", subcore_axis_name="s")
  k = pl.kernel(
      _body,
      out_type=(
          jax.ShapeDtypeStruct((_TOTAL_W, WORD_D), jnp.float32),
          jax.ShapeDtypeStruct((B, TOPIC_D), jnp.float32),
          jax.ShapeDtypeStruct((B, ACT_D), jnp.float32),
      ),
      mesh=mesh,
      scratch_types=(
          pltpu.VMEM((_ROWS_PER_W, _GCHUNK), jnp.int32),
          pltpu.VMEM((_SUPER, WORD_D), jnp.float32),
          pltpu.VMEM((_SUPER, WORD_D), jnp.float32),
          pltpu.VMEM((_S_PER,), jnp.int32),
          pltpu.VMEM((_S_PER, TOPIC_D), jnp.float32),
          pltpu.VMEM((_S_PER,), jnp.int32),
          pltpu.VMEM((_S_PER, ACT_D), jnp.float32),
          pltpu.SemaphoreType.DMA,
          pltpu.SemaphoreType.DMA,
          pltpu.SemaphoreType.DMA,
      ),
      compiler_params=pltpu.CompilerParams(use_tc_tiling_on_sc=False),
  )
  return k(word_ids2d, topic_ids, act_ids, word_table, topic_table, act_table)


def kernel(word_ids, topic_ids, act_ids, word_table, topic_table, act_table):
  word_ids2d = word_ids.reshape(_TOTAL_W // _GCHUNK, _GCHUNK).astype(jnp.int32)
  wout, tout, aout = _run(word_ids2d,
                          topic_ids.astype(jnp.int32),
                          act_ids.astype(jnp.int32),
                          word_table, topic_table, act_table)
  return (wout.reshape(B, L, WORD_D), tout, aout)
